# Initial kernel scaffold; baseline (speedup 1.0000x reference)
import jax
import jax.numpy as jnp
from jax.experimental import pallas as pl


def kernel(x):
    raise NotImplementedError("write your pallas kernel here")


# speedup vs baseline: 1.1529x; 1.1529x over previous
import math

import jax
import jax.numpy as jnp
from jax.experimental import pallas as pl
from jax.experimental.pallas import tpu as pltpu

_HIDDEN = 128


def _pe_weights(C, hidden):
    block = {2: 16, 4: 64}[C]
    j = jnp.arange(hidden)
    b = j % block
    L = b // (2 * C)
    within = b % (2 * C)
    coord = within // 2
    is_cos = (within % 2) == 1
    scale = (2.0 ** L.astype(jnp.float32)) * math.pi
    W = (coord[None, :] == jnp.arange(C)[:, None]).astype(jnp.float32) * scale[None, :]
    phase = jnp.where(is_cos, 0.5 * math.pi, 0.0).astype(jnp.float32)[None, :]
    return W, phase


def _pe_kernel(x_ref, w_ref, p_ref, o_ref):
    arg = jnp.dot(x_ref[...], w_ref[...], preferred_element_type=jnp.float32)
    o_ref[...] = jnp.sin(arg + p_ref[...])


def _round_up(a, b):
    return ((a + b - 1) // b) * b


def kernel(x):
    input_shape = x.shape
    C = input_shape[-1]
    hidden = _HIDDEN

    x2 = x.reshape(-1, C).astype(jnp.float32)
    N = x2.shape[0]

    W, phase = _pe_weights(C, hidden)

    TB = max(8, min(8192, (4 * 1024 * 1024) // (hidden * 4)))
    TB = min(TB, _round_up(-(-N // 2), 8), _round_up(N, 8))

    grid = (pl.cdiv(N, TB),)

    cost = pl.CostEstimate(
        flops=int(2 * N * C * hidden + N * hidden),
        transcendentals=int(N * hidden),
        bytes_accessed=int(N * C * 4 + N * hidden * 4),
    )

    out = pl.pallas_call(
        _pe_kernel,
        out_shape=jax.ShapeDtypeStruct((N, hidden), jnp.float32),
        grid=grid,
        in_specs=[
            pl.BlockSpec((TB, C), lambda i: (i, 0)),
            pl.BlockSpec((C, hidden), lambda i: (0, 0)),
            pl.BlockSpec((1, hidden), lambda i: (0, 0)),
        ],
        out_specs=pl.BlockSpec((TB, hidden), lambda i: (i, 0)),
        compiler_params=pltpu.CompilerParams(
            dimension_semantics=("parallel",),
            vmem_limit_bytes=64 * 1024 * 1024,
        ),
        cost_estimate=cost,
    )(x2, W, phase)

    return out.reshape(tuple(input_shape[:-1]) + (hidden,))


# speedup vs baseline: 2.2644x; 1.9641x over previous
import math

import jax
import jax.numpy as jnp
from jax.experimental import pallas as pl
from jax.experimental.pallas import tpu as pltpu

_HIDDEN = 128


def _pe_weights(C, hidden):
    block = {2: 16, 4: 64}[C]
    j = jnp.arange(hidden)
    b = j % block
    L = b // (2 * C)
    within = b % (2 * C)
    coord = within // 2
    is_cos = (within % 2) == 1
    scale = 2.0 ** L.astype(jnp.float32)
    W = (coord[None, :] == jnp.arange(C)[:, None]).astype(jnp.float32) * scale[None, :]
    phase = jnp.where(is_cos, 0.5, 0.0).astype(jnp.float32)[None, :]
    return W, phase


_S1 = 3.141590269254692
_S3 = -5.167406695260764
_S5 = 2.544010154106732
_S7 = -0.5594614847235516


def _sin_pi(t):
    kf = jax.lax.round(t, jax.lax.RoundingMethod.TO_NEAREST_EVEN)
    r = t - kf
    parity = jax.lax.bitwise_and(kf.astype(jnp.int32), 1)
    signbit = jax.lax.shift_left(parity, 31)
    r2 = r * r
    p = _S7 * r2 + _S5
    p = p * r2 + _S3
    p = p * r2 + _S1
    s = r * p
    return jax.lax.bitcast_convert_type(
        jax.lax.bitcast_convert_type(s, jnp.int32) ^ signbit, jnp.float32)


def _pe_kernel(x_ref, w_ref, p_ref, o_ref):
    t = jnp.dot(x_ref[...], w_ref[...], preferred_element_type=jnp.float32)
    o_ref[...] = _sin_pi(t + p_ref[...])


def _round_up(a, b):
    return ((a + b - 1) // b) * b


def kernel(x):
    input_shape = x.shape
    C = input_shape[-1]
    hidden = _HIDDEN

    x2 = x.reshape(-1, C).astype(jnp.float32)
    N = x2.shape[0]

    W, phase = _pe_weights(C, hidden)

    TB = max(8, min(8192, (4 * 1024 * 1024) // (hidden * 4)))
    TB = min(TB, _round_up(-(-N // 2), 8), _round_up(N, 8))

    grid = (pl.cdiv(N, TB),)

    cost = pl.CostEstimate(
        flops=int(2 * N * C * hidden + N * hidden),
        transcendentals=int(N * hidden),
        bytes_accessed=int(N * C * 4 + N * hidden * 4),
    )

    out = pl.pallas_call(
        _pe_kernel,
        out_shape=jax.ShapeDtypeStruct((N, hidden), jnp.float32),
        grid=grid,
        in_specs=[
            pl.BlockSpec((TB, C), lambda i: (i, 0)),
            pl.BlockSpec((C, hidden), lambda i: (0, 0)),
            pl.BlockSpec((1, hidden), lambda i: (0, 0)),
        ],
        out_specs=pl.BlockSpec((TB, hidden), lambda i: (i, 0)),
        compiler_params=pltpu.CompilerParams(
            dimension_semantics=("parallel",),
            vmem_limit_bytes=64 * 1024 * 1024,
        ),
        cost_estimate=cost,
    )(x2, W, phase)

    return out.reshape(tuple(input_shape[:-1]) + (hidden,))


# speedup vs baseline: 10.2827x; 4.5411x over previous
import jax
import jax.numpy as jnp
from jax.experimental import pallas as pl
from jax.experimental.pallas import tpu as pltpu

_HIDDEN = 128


def _pe_weights(C, hidden):
    block = {2: 16, 4: 64}[C]
    j = jnp.arange(hidden)
    b = j % block
    L = b // (2 * C)
    within = b % (2 * C)
    coord = within // 2
    is_cos = (within % 2) == 1
    scale = 2.0 ** L.astype(jnp.float32)
    W = (coord[None, :] == jnp.arange(C)[:, None]).astype(jnp.float32) * scale[None, :]
    phase = jnp.where(is_cos, 0.5, 0.0).astype(jnp.float32)[None, :]
    return W, phase


_S1 = 3.141590269254692
_S3 = -5.167406695260764
_S5 = 2.544010154106732
_S7 = -0.5594614847235516


def _sin_pi(t):
    kf = jax.lax.round(t, jax.lax.RoundingMethod.TO_NEAREST_EVEN)
    r = t - kf
    parity = jax.lax.bitwise_and(kf.astype(jnp.int32), 1)
    signbit = jax.lax.shift_left(parity, 31)
    r2 = r * r
    p = _S7 * r2 + _S5
    p = p * r2 + _S3
    p = p * r2 + _S1
    s = r * p
    return jax.lax.bitcast_convert_type(
        jax.lax.bitcast_convert_type(s, jnp.int32) ^ signbit, jnp.float32)


def _make_pe_kernel(a_blk, C, M):
    def _pe_kernel(x_ref, w_ref, p_ref, o_ref):
        phase = p_ref[...]
        w = w_ref[...]
        for a in range(a_blk):
            u = x_ref[a * C:(a + 1) * C, :]
            t = jax.lax.dot_general(
                u, w, (((0,), (0,)), ((), ())),
                preferred_element_type=jnp.float32)
            o_ref[a * M:(a + 1) * M, :] = _sin_pi(t + phase)
    return _pe_kernel


def kernel(x):
    input_shape = x.shape
    C = input_shape[-1]
    hidden = _HIDDEN

    if x.ndim < 3:
        x3 = x.reshape(1, -1, C)
    else:
        x3 = x.reshape(-1, input_shape[-2], C)
    x3 = x3.astype(jnp.float32)
    A, M, _ = x3.shape
    N = A * M

    xt = jnp.swapaxes(x3, 1, 2).reshape(A * C, M)

    W, phase = _pe_weights(C, hidden)

    A_BLK = max(1, min(A, (4 * 1024 * 1024) // (M * hidden * 4)))
    while A % A_BLK:
        A_BLK -= 1

    grid = (A // A_BLK,)

    cost = pl.CostEstimate(
        flops=int(2 * N * C * hidden + N * hidden),
        transcendentals=int(N * hidden),
        bytes_accessed=int(N * C * 4 + N * hidden * 4),
    )

    out = pl.pallas_call(
        _make_pe_kernel(A_BLK, C, M),
        out_shape=jax.ShapeDtypeStruct((N, hidden), jnp.float32),
        grid=grid,
        in_specs=[
            pl.BlockSpec((A_BLK * C, M), lambda i: (i, 0)),
            pl.BlockSpec((C, hidden), lambda i: (0, 0)),
            pl.BlockSpec((1, hidden), lambda i: (0, 0)),
        ],
        out_specs=pl.BlockSpec((A_BLK * M, hidden), lambda i: (i, 0)),
        compiler_params=pltpu.CompilerParams(
            dimension_semantics=("parallel",),
            vmem_limit_bytes=64 * 1024 * 1024,
        ),
        cost_estimate=cost,
    )(xt, W, phase)

    return out.reshape(tuple(input_shape[:-1]) + (hidden,))


# speedup vs baseline: 15.4469x; 1.5022x over previous
import jax
import jax.numpy as jnp
from jax.experimental import pallas as pl
from jax.experimental.pallas import tpu as pltpu

_HIDDEN = 128


def _pe_weights(C, hidden, G):
    block = {2: 16, 4: 64}[C]
    b = jnp.arange(block)
    L = b // (2 * C)
    within = b % (2 * C)
    coord = within // 2
    is_cos = (within % 2) == 1
    scale = 2.0 ** L.astype(jnp.float32)
    B = (coord[None, :] == jnp.arange(C)[:, None]).astype(jnp.float32) * scale[None, :]
    Wu = jnp.kron(jnp.eye(G, dtype=jnp.float32), B)
    phase = jnp.tile(jnp.where(is_cos, 0.5, 0.0).astype(jnp.float32), G)[None, :]
    rep = hidden // block
    R1 = jnp.tile(jnp.eye(block, dtype=jnp.float32), (1, rep))
    R = jnp.kron(jnp.eye(G, dtype=jnp.float32), R1)
    return Wu, phase, R


_S1 = 3.141590269254692
_S3 = -5.167406695260764
_S5 = 2.544010154106732
_S7 = -0.5594614847235516


def _sin_pi(t):
    kf = jax.lax.round(t, jax.lax.RoundingMethod.TO_NEAREST_EVEN)
    r = t - kf
    parity = jax.lax.bitwise_and(kf.astype(jnp.int32), 1)
    signbit = jax.lax.shift_left(parity, 31)
    r2 = r * r
    p = _S7 * r2 + _S5
    p = p * r2 + _S3
    p = p * r2 + _S1
    s = r * p
    return jax.lax.bitcast_convert_type(
        jax.lax.bitcast_convert_type(s, jnp.int32) ^ signbit, jnp.float32)


def _make_pe_kernel(G, M, hidden):
    def _pe_kernel(x_ref, w_ref, p_ref, r_ref, o_ref):
        t = jax.lax.dot_general(
            x_ref[...], w_ref[...], (((0,), (0,)), ((), ())),
            preferred_element_type=jnp.float32)
        s = _sin_pi(t + p_ref[...])
        for g in range(G):
            o_ref[g * M:(g + 1) * M, :] = jnp.dot(
                s, r_ref[:, g * hidden:(g + 1) * hidden],
                preferred_element_type=jnp.float32)
    return _pe_kernel


def kernel(x):
    input_shape = x.shape
    C = input_shape[-1]
    hidden = _HIDDEN
    block = {2: 16, 4: 64}[C]
    G = 128 // block

    if x.ndim < 3:
        x3 = x.reshape(1, -1, C)
    else:
        x3 = x.reshape(-1, input_shape[-2], C)
    x3 = x3.astype(jnp.float32)
    A, M, _ = x3.shape
    N = A * M

    xt = jnp.swapaxes(x3, 1, 2).reshape(A * C, M)

    if A % G:
        pad = G - A % G
        xt = jnp.concatenate([xt, jnp.zeros((pad * C, M), jnp.float32)], axis=0)
        A += pad

    Wu, phase, R = _pe_weights(C, hidden, G)

    grid = (A // G,)

    cost = pl.CostEstimate(
        flops=int(2 * N * C * block + 2 * N * block * hidden),
        transcendentals=int(N * block),
        bytes_accessed=int(N * C * 4 + N * hidden * 4),
    )

    out = pl.pallas_call(
        _make_pe_kernel(G, M, hidden),
        out_shape=jax.ShapeDtypeStruct((A * M, hidden), jnp.float32),
        grid=grid,
        in_specs=[
            pl.BlockSpec((G * C, M), lambda i: (i, 0)),
            pl.BlockSpec((G * C, G * block), lambda i: (0, 0)),
            pl.BlockSpec((1, G * block), lambda i: (0, 0)),
            pl.BlockSpec((G * block, G * hidden), lambda i: (0, 0)),
        ],
        out_specs=pl.BlockSpec((G * M, hidden), lambda i: (i, 0)),
        compiler_params=pltpu.CompilerParams(
            dimension_semantics=("parallel",),
            vmem_limit_bytes=64 * 1024 * 1024,
        ),
        cost_estimate=cost,
    )(xt, Wu, phase, R)

    return out[:N].reshape(tuple(input_shape[:-1]) + (hidden,))


# speedup vs baseline: 18.1340x; 1.1740x over previous
import jax
import jax.numpy as jnp
from jax.experimental import pallas as pl
from jax.experimental.pallas import tpu as pltpu

_HIDDEN = 128


def _pe_weights(C, hidden, G):
    block = {2: 16, 4: 64}[C]
    b = jnp.arange(block)
    L = b // (2 * C)
    within = b % (2 * C)
    coord = within // 2
    is_cos = (within % 2) == 1
    scale = 2.0 ** L.astype(jnp.float32)
    B = (coord[None, :] == jnp.arange(C)[:, None]).astype(jnp.float32) * scale[None, :]
    Wu = jnp.kron(jnp.eye(G, dtype=jnp.float32), B)
    phase = jnp.tile(jnp.where(is_cos, 0.5, 0.0).astype(jnp.float32), G)[None, :]
    rep = hidden // block
    R1 = jnp.tile(jnp.eye(block, dtype=jnp.float32), (1, rep))
    R = jnp.kron(jnp.eye(G, dtype=jnp.float32), R1)
    return Wu, phase, R


_S1 = 3.141590269254692
_S3 = -5.167406695260764
_S5 = 2.544010154106732
_S7 = -0.5594614847235516


def _sin_pi(t):
    kf = jax.lax.round(t, jax.lax.RoundingMethod.TO_NEAREST_EVEN)
    r = t - kf
    parity = jax.lax.bitwise_and(kf.astype(jnp.int32), 1)
    signbit = jax.lax.shift_left(parity, 31)
    r2 = r * r
    p = _S7 * r2 + _S5
    p = p * r2 + _S3
    p = p * r2 + _S1
    s = r * p
    return jax.lax.bitcast_convert_type(
        jax.lax.bitcast_convert_type(s, jnp.int32) ^ signbit, jnp.float32)


def _make_pe_kernel(G, M, hidden, C, P):
    def _pe_kernel(x_ref, w_ref, p_ref, r_ref, o_ref):
        phase = p_ref[...]
        w = w_ref[...]
        for p in range(P):
            u = x_ref[p * G * C:(p + 1) * G * C, :]
            t = jax.lax.dot_general(
                u, w, (((0,), (0,)), ((), ())),
                preferred_element_type=jnp.float32)
            s = _sin_pi(t + phase)
            for g in range(G):
                o_ref[(p * G + g) * M:(p * G + g + 1) * M, :] = jnp.dot(
                    s, r_ref[:, g * hidden:(g + 1) * hidden],
                    preferred_element_type=jnp.float32)
    return _pe_kernel


def kernel(x):
    input_shape = x.shape
    C = input_shape[-1]
    hidden = _HIDDEN
    block = {2: 16, 4: 64}[C]
    G = 128 // block

    if x.ndim < 3:
        x3 = x.reshape(1, -1, C)
    else:
        x3 = x.reshape(-1, input_shape[-2], C)
    x3 = x3.astype(jnp.float32)
    A, M, _ = x3.shape
    N = A * M

    xt = jnp.swapaxes(x3, 1, 2).reshape(A * C, M)

    P = 2
    AB = P * G
    if A % AB:
        pad = AB - A % AB
        xt = jnp.concatenate([xt, jnp.zeros((pad * C, M), jnp.float32)], axis=0)
        A += pad

    Wu, phase, R = _pe_weights(C, hidden, G)

    grid = (A // AB,)

    cost = pl.CostEstimate(
        flops=int(2 * N * C * block + 2 * N * block * hidden),
        transcendentals=int(N * block),
        bytes_accessed=int(N * C * 4 + N * hidden * 4),
    )

    out = pl.pallas_call(
        _make_pe_kernel(G, M, hidden, C, P),
        out_shape=jax.ShapeDtypeStruct((A * M, hidden), jnp.float32),
        grid=grid,
        in_specs=[
            pl.BlockSpec((AB * C, M), lambda i: (i, 0)),
            pl.BlockSpec((G * C, G * block), lambda i: (0, 0)),
            pl.BlockSpec((1, G * block), lambda i: (0, 0)),
            pl.BlockSpec((G * block, G * hidden), lambda i: (0, 0)),
        ],
        out_specs=pl.BlockSpec((AB * M, hidden), lambda i: (i, 0)),
        compiler_params=pltpu.CompilerParams(
            dimension_semantics=("parallel",),
            vmem_limit_bytes=64 * 1024 * 1024,
        ),
        cost_estimate=cost,
    )(xt, Wu, phase, R)

    return out[:N].reshape(tuple(input_shape[:-1]) + (hidden,))


# speedup vs baseline: 18.2500x; 1.0064x over previous
import jax
import jax.numpy as jnp
from jax.experimental import pallas as pl
from jax.experimental.pallas import tpu as pltpu

_HIDDEN = 128


def _pe_weights(C, hidden, G):
    block = {2: 16, 4: 64}[C]
    b = jnp.arange(block)
    L = b // (2 * C)
    within = b % (2 * C)
    coord = within // 2
    is_cos = (within % 2) == 1
    scale = 2.0 ** L.astype(jnp.float32)
    B = (coord[None, :] == jnp.arange(C)[:, None]).astype(jnp.float32) * scale[None, :]
    Wu = jnp.kron(jnp.eye(G, dtype=jnp.float32), B)
    phase = jnp.tile(jnp.where(is_cos, 0.5, 0.0).astype(jnp.float32), G)[None, :]
    rep = hidden // block
    R1 = jnp.tile(jnp.eye(block, dtype=jnp.float32), (1, rep))
    R = jnp.kron(jnp.eye(G, dtype=jnp.float32), R1)
    return Wu, phase, R


_S1 = 3.141590269254692
_S3 = -5.167406695260764
_S5 = 2.544010154106732
_S7 = -0.5594614847235516


def _sin_pi(t):
    kf = jax.lax.round(t, jax.lax.RoundingMethod.TO_NEAREST_EVEN)
    r = t - kf
    parity = jax.lax.bitwise_and(kf.astype(jnp.int32), 1)
    signbit = jax.lax.shift_left(parity, 31)
    r2 = r * r
    p = _S7 * r2 + _S5
    p = p * r2 + _S3
    p = p * r2 + _S1
    s = r * p
    return jax.lax.bitcast_convert_type(
        jax.lax.bitcast_convert_type(s, jnp.int32) ^ signbit, jnp.float32)


def _make_pe_kernel(G, M, hidden, C, P):
    def _pe_kernel(x_ref, w_ref, p_ref, r_ref, o_ref):
        phase = p_ref[...]
        w = w_ref[...]
        for p in range(P):
            u = x_ref[p * G * C:(p + 1) * G * C, :]
            t = jax.lax.dot_general(
                u, w, (((0,), (0,)), ((), ())),
                preferred_element_type=jnp.float32)
            s = _sin_pi(t + phase)
            for g in range(G):
                o_ref[(p * G + g) * M:(p * G + g + 1) * M, :] = jnp.dot(
                    s, r_ref[:, g * hidden:(g + 1) * hidden],
                    preferred_element_type=jnp.float32)
    return _pe_kernel


def kernel(x):
    input_shape = x.shape
    C = input_shape[-1]
    hidden = _HIDDEN
    block = {2: 16, 4: 64}[C]
    G = 128 // block

    if x.ndim < 3:
        x3 = x.reshape(1, -1, C)
    else:
        x3 = x.reshape(-1, input_shape[-2], C)
    x3 = x3.astype(jnp.float32)
    A, M, _ = x3.shape
    N = A * M

    xt = jnp.swapaxes(x3, 1, 2).reshape(A * C, M)

    P = 4
    AB = P * G
    if A % AB:
        pad = AB - A % AB
        xt = jnp.concatenate([xt, jnp.zeros((pad * C, M), jnp.float32)], axis=0)
        A += pad

    Wu, phase, R = _pe_weights(C, hidden, G)

    grid = (A // AB,)

    cost = pl.CostEstimate(
        flops=int(2 * N * C * block + 2 * N * block * hidden),
        transcendentals=int(N * block),
        bytes_accessed=int(N * C * 4 + N * hidden * 4),
    )

    out = pl.pallas_call(
        _make_pe_kernel(G, M, hidden, C, P),
        out_shape=jax.ShapeDtypeStruct((A * M, hidden), jnp.float32),
        grid=grid,
        in_specs=[
            pl.BlockSpec((AB * C, M), lambda i: (i, 0)),
            pl.BlockSpec((G * C, G * block), lambda i: (0, 0)),
            pl.BlockSpec((1, G * block), lambda i: (0, 0)),
            pl.BlockSpec((G * block, G * hidden), lambda i: (0, 0)),
        ],
        out_specs=pl.BlockSpec((AB * M, hidden), lambda i: (i, 0)),
        compiler_params=pltpu.CompilerParams(
            dimension_semantics=("parallel",),
            vmem_limit_bytes=64 * 1024 * 1024,
        ),
        cost_estimate=cost,
    )(xt, Wu, phase, R)

    return out[:N].reshape(tuple(input_shape[:-1]) + (hidden,))
